# Initial kernel scaffold; baseline (speedup 1.0000x reference)
import jax
import jax.numpy as jnp
from jax.experimental import pallas as pl


def kernel(input, weight):
    raise NotImplementedError("write your pallas kernel here")


# speedup vs baseline: 1.8631x; 1.8631x over previous
import functools

import jax
import jax.numpy as jnp
from jax import lax
from jax.experimental import pallas as pl
from jax.experimental.pallas import tpu as pltpu
from jax.experimental.pallas import tpu_sc as plsc

D = 64
B_TOTAL = 16384 * 50
NW = 32
B_PER_W = B_TOTAL // NW
CHUNK = 640
N_CHUNKS = B_PER_W // CHUNK
N_PAIRS = N_CHUNKS // 2


def _sc_embedding_gather(idx_flat, weight):
    mesh = plsc.VectorSubcoreMesh(core_axis_name="c", subcore_axis_name="s")

    @functools.partial(
        pl.kernel,
        mesh=mesh,
        out_type=jax.ShapeDtypeStruct((B_TOTAL, D), jnp.float32),
        scratch_types=[
            pltpu.VMEM((B_PER_W,), jnp.int32),
            pltpu.VMEM((CHUNK, D), jnp.float32),
            pltpu.VMEM((CHUNK, D), jnp.float32),
            pltpu.SemaphoreType.DMA,
            pltpu.SemaphoreType.DMA,
            pltpu.SemaphoreType.DMA,
            pltpu.SemaphoreType.DMA,
        ],
        compiler_params=pltpu.CompilerParams(use_tc_tiling_on_sc=False),
    )
    def k(table_hbm, idx_hbm, out_hbm, idx_v, rows_a, rows_b, sem_ga,
          sem_gb, sem_sa, sem_sb):
        wid = lax.axis_index("s") * 2 + lax.axis_index("c")
        base = wid * B_PER_W

        pltpu.sync_copy(idx_hbm.at[pl.ds(base, B_PER_W)], idx_v)

        def gather(g, rows, sem):
            pltpu.async_copy(
                table_hbm.at[idx_v.at[pl.ds(g * CHUNK, CHUNK)]], rows, sem)

        def wait_gather(g, rows, sem):
            pltpu.make_async_copy(
                table_hbm.at[idx_v.at[pl.ds(g * CHUNK, CHUNK)]], rows, sem
            ).wait()

        def store(g, rows, sem):
            pltpu.async_copy(
                rows, out_hbm.at[pl.ds(base + g * CHUNK, CHUNK)], sem)

        def wait_store(g, rows, sem):
            pltpu.make_async_copy(
                rows, out_hbm.at[pl.ds(base + g * CHUNK, CHUNK)], sem
            ).wait()

        gather(0, rows_a, sem_ga)
        gather(1, rows_b, sem_gb)

        def body(i, carry):
            g0 = 2 * i
            g1 = g0 + 1
            wait_gather(g0, rows_a, sem_ga)
            store(g0, rows_a, sem_sa)
            wait_gather(g1, rows_b, sem_gb)
            store(g1, rows_b, sem_sb)
            wait_store(g0, rows_a, sem_sa)

            @pl.when(i + 1 < N_PAIRS)
            def _():
                gather(g0 + 2, rows_a, sem_ga)

            wait_store(g1, rows_b, sem_sb)

            @pl.when(i + 1 < N_PAIRS)
            def _():
                gather(g1 + 2, rows_b, sem_gb)

            return carry

        lax.fori_loop(0, N_PAIRS, body, 0)

    return k(weight, idx_flat)


def kernel(input, weight):
    idx_flat = input.reshape(-1).astype(jnp.int32)
    out = _sc_embedding_gather(idx_flat, weight)
    return out.reshape(input.shape + (D,))


# speedup vs baseline: 1.8640x; 1.0005x over previous
import functools

import jax
import jax.numpy as jnp
from jax import lax
from jax.experimental import pallas as pl
from jax.experimental.pallas import tpu as pltpu
from jax.experimental.pallas import tpu_sc as plsc

D = 64
NUM_ROWS = 1000000
NUM_ROW_PAIRS = NUM_ROWS // 2
B_TOTAL = 16384 * 50
NW = 32
B_PER_W = B_TOTAL // NW
CHUNK = 640
N_CHUNKS = B_PER_W // CHUNK
N_PAIRS = N_CHUNKS // 2


def _sc_embedding_gather(idx_flat, weight):
    mesh = plsc.VectorSubcoreMesh(core_axis_name="c", subcore_axis_name="s")

    @functools.partial(
        pl.kernel,
        mesh=mesh,
        out_type=jax.ShapeDtypeStruct((B_TOTAL, D), jnp.float32),
        scratch_types=[
            pltpu.VMEM((B_PER_W,), jnp.int32),
            pltpu.VMEM((CHUNK, D), jnp.float32),
            pltpu.VMEM((CHUNK, D), jnp.float32),
            pltpu.SemaphoreType.DMA,
            pltpu.SemaphoreType.DMA,
            pltpu.SemaphoreType.DMA,
            pltpu.SemaphoreType.DMA,
        ],
        compiler_params=pltpu.CompilerParams(use_tc_tiling_on_sc=False),
    )
    def k(table_hbm, idx_hbm, out_hbm, idx_v, rows_a, rows_b, sem_ga,
          sem_gb, sem_sa, sem_sb):
        wid = lax.axis_index("s") * 2 + lax.axis_index("c")
        base = wid * B_PER_W

        pltpu.sync_copy(idx_hbm.at[pl.ds(base, B_PER_W)], idx_v)

        def gather(g, rows, sem):
            pltpu.async_copy(
                table_hbm.at[idx_v.at[pl.ds(g * CHUNK, CHUNK)]], rows, sem)

        def wait_gather(g, rows, sem):
            pltpu.make_async_copy(
                table_hbm.at[idx_v.at[pl.ds(g * CHUNK, CHUNK)]], rows, sem
            ).wait()

        def store(g, rows, sem):
            pltpu.async_copy(
                rows, out_hbm.at[pl.ds(base + g * CHUNK, CHUNK)], sem)

        def wait_store(g, rows, sem):
            pltpu.make_async_copy(
                rows, out_hbm.at[pl.ds(base + g * CHUNK, CHUNK)], sem
            ).wait()

        gather(0, rows_a, sem_ga)
        gather(1, rows_b, sem_gb)

        def body(i, carry):
            g0 = 2 * i
            g1 = g0 + 1
            wait_gather(g0, rows_a, sem_ga)
            store(g0, rows_a, sem_sa)
            wait_gather(g1, rows_b, sem_gb)
            store(g1, rows_b, sem_sb)
            wait_store(g0, rows_a, sem_sa)

            @pl.when(i + 1 < N_PAIRS)
            def _():
                gather(g0 + 2, rows_a, sem_ga)

            wait_store(g1, rows_b, sem_sb)

            @pl.when(i + 1 < N_PAIRS)
            def _():
                gather(g1 + 2, rows_b, sem_gb)

            return carry

        lax.fori_loop(0, N_PAIRS, body, 0)

    return k(weight, idx_flat)


def kernel(input, weight):
    idx_flat = input.reshape(-1).astype(jnp.int32)
    w2 = lax.optimization_barrier(jnp.reshape(weight, (NUM_ROW_PAIRS, 2 * D)))
    w3 = jnp.reshape(w2, (NUM_ROWS, D))
    out = _sc_embedding_gather(idx_flat, w3)
    return out.reshape(input.shape + (D,))
